# baseline (speedup 1.0000x reference)
import jax
import jax.numpy as jnp
from jax import lax
from jax.experimental import pallas as pl
from jax.experimental.pallas import tpu as pltpu
from jax.experimental.pallas import tpu_sc as plsc

NC = 2
NS = 16
NW = NC * NS
NUM_HYPEREDGES = 10000


def _sc_propagate(table, gidx, sidx, n_out, with_deg, chunk, nb=3, ahead=1):
    E = gidx.shape[0]
    T, F = table.shape
    assert E % (NW * chunk) == 0
    ept = E // NW
    full = ept // chunk
    rpt = -(-n_out // NS // 8) * 8
    n_pad = NS * rpt
    assert n_pad > n_out
    dpt = -(-T // NS // 16) * 16 if with_deg else 0
    bpt = -(-(n_out + 1) // NS // 16) * 16 if with_deg else 0
    t_pad, b_pad = NS * dpt, NS * bpt
    zlen = max(dpt, bpt)

    out_type = [jax.ShapeDtypeStruct((NC, n_pad, F), jnp.float32)]
    if with_deg:
        out_type += [jax.ShapeDtypeStruct((NC * t_pad,), jnp.float32),
                     jax.ShapeDtypeStruct((NC * b_pad,), jnp.float32)]

    NB = nb
    scratch = []
    for _ in range(NB):
        scratch += [pltpu.VMEM((chunk,), jnp.int32),
                    pltpu.VMEM((chunk,), jnp.int32),
                    pltpu.VMEM((chunk, F), jnp.float32),
                    pltpu.SemaphoreType.DMA,
                    pltpu.SemaphoreType.DMA]
    scratch.append(pltpu.VMEM_SHARED((n_pad, F), jnp.float32))
    if with_deg:
        scratch += [pltpu.VMEM((chunk,), jnp.float32),
                    pltpu.VMEM((zlen,), jnp.float32),
                    pltpu.VMEM_SHARED((t_pad,), jnp.float32),
                    pltpu.VMEM_SHARED((b_pad,), jnp.float32)]

    mesh = plsc.VectorSubcoreMesh(core_axis_name="c", subcore_axis_name="s",
                                  num_cores=NC, num_subcores=NS)

    def body(*refs):
        it = iter(refs)
        table_r, gidx_r, sidx_r, z2_r = (next(it) for _ in range(4))
        p_r = next(it)
        if with_deg:
            dg_r, bg_r = next(it), next(it)
        bufs = tuple(tuple(next(it) for _ in range(5)) for _ in range(NB))
        acc = next(it)
        if with_deg:
            ones, zbuf, acc_d, acc_b = next(it), next(it), next(it), next(it)

        c = lax.axis_index("c")
        s = lax.axis_index("s")
        wid = c * NS + s
        base = wid * ept

        pltpu.sync_copy(z2_r, acc.at[pl.ds(s * rpt, rpt)])
        if with_deg:
            for i in range(zlen // 16):
                zbuf[pl.ds(i * 16, 16)] = jnp.zeros((16,), jnp.float32)
            pltpu.sync_copy(zbuf.at[pl.ds(0, dpt)], acc_d.at[pl.ds(s * dpt, dpt)])
            pltpu.sync_copy(zbuf.at[pl.ds(0, bpt)], acc_b.at[pl.ds(s * bpt, bpt)])
            for i in range(chunk // 16):
                ones[pl.ds(i * 16, 16)] = jnp.ones((16,), jnp.float32)
        plsc.subcore_barrier()

        def load_idx(j, gvx, svx):
            off = pl.multiple_of(base + j * chunk, 8)
            pltpu.sync_copy(gidx_r.at[pl.ds(off, chunk)], gvx)
            pltpu.sync_copy(sidx_r.at[pl.ds(off, chunk)], svx)

        A = ahead

        def step(j, bx, bz):
            gvx, svx, rowsx, gsemx, ssemx = bx
            gvz, svz, rowsz, gsemz, ssemz = bz

            @pl.when(j >= NB - A)
            def _():
                pltpu.make_async_copy(rowsz, acc.at[svz], ssemz).wait()

            @pl.when(j + A < full)
            def _():
                load_idx(j + A, gvz, svz)
                pltpu.async_copy(table_r.at[gvz], rowsz, gsemz)

            pltpu.make_async_copy(table_r.at[gvx], rowsx, gsemx).wait()

            pltpu.async_copy(rowsx, acc.at[svx], ssemx, add=True)
            if with_deg:
                pltpu.sync_copy(ones, acc_d.at[gvx], add=True)
                pltpu.sync_copy(ones, acc_b.at[svx], add=True)

        for i in range(min(A, full)):
            load_idx(i, bufs[i][0], bufs[i][1])
            pltpu.async_copy(table_r.at[bufs[i][0]], bufs[i][2], bufs[i][3])

        def loop_body(k, carry):
            j = NB * k
            for t in range(NB):
                step(j + t, bufs[t], bufs[(t + A) % NB])
            return carry

        lax.fori_loop(0, full // NB, loop_body, 0)
        for t in range(full - NB * (full // NB)):
            j = NB * (full // NB) + t
            step(jnp.int32(j), bufs[j % NB], bufs[(j + A) % NB])

        for t in range(max(0, full - (NB - A)), full):
            _gv, svL, rowsL, _gs, ssemL = bufs[t % NB]
            pltpu.make_async_copy(rowsL, acc.at[svL], ssemL).wait()

        plsc.subcore_barrier()
        pltpu.sync_copy(acc.at[pl.ds(s * rpt, rpt)], p_r.at[c, pl.ds(s * rpt, rpt)])
        if with_deg:
            pltpu.sync_copy(acc_d.at[pl.ds(s * dpt, dpt)], zbuf.at[pl.ds(0, dpt)])
            pltpu.sync_copy(zbuf.at[pl.ds(0, dpt)],
                            dg_r.at[pl.ds(pl.multiple_of(c * t_pad + s * dpt, 8), dpt)])
            pltpu.sync_copy(acc_b.at[pl.ds(s * bpt, bpt)], zbuf.at[pl.ds(0, bpt)])
            pltpu.sync_copy(zbuf.at[pl.ds(0, bpt)],
                            bg_r.at[pl.ds(pl.multiple_of(c * b_pad + s * bpt, 8), bpt)])

    args = [table, gidx, sidx, jnp.zeros((rpt, F), jnp.float32)]

    f = pl.kernel(body, out_type=tuple(out_type), mesh=mesh,
                  scratch_types=scratch,
                  compiler_params=pltpu.CompilerParams(use_tc_tiling_on_sc=False))
    out = f(*args)
    return list(out) if isinstance(out, (tuple, list)) else [out]


_BLK = 1000


def _tc_mm(x, w):
    n, f = x.shape
    f2 = w.shape[1]

    def body(x_r, w_r, o_r):
        o_r[...] = jnp.dot(x_r[...], w_r[...], preferred_element_type=jnp.float32)

    return pl.pallas_call(
        body,
        grid=(n // _BLK,),
        in_specs=[pl.BlockSpec((_BLK, f), lambda i: (i, 0)),
                  pl.BlockSpec((f, f2), lambda i: (0, 0))],
        out_specs=pl.BlockSpec((_BLK, f2), lambda i: (i, 0)),
        out_shape=jax.ShapeDtypeStruct((n, f2), jnp.float32),
    )(x, w)


def _tc_combine_scale(p, degp, dcorr, bias=None):
    f = p.shape[2]
    n = degp.shape[1]

    def body(p_r, d_r, c_r, *rest):
        o_r = rest[-1]
        ssum = p_r[0] + p_r[1]
        deg = d_r[0] + d_r[1] - c_r[...]
        o = ssum / jnp.maximum(deg, 1.0)
        if bias is not None:
            o = o + rest[0][...]
        o_r[...] = o

    in_specs = [pl.BlockSpec((NC, _BLK, f), lambda i: (0, i, 0)),
                pl.BlockSpec((NC, _BLK, 1), lambda i: (0, i, 0)),
                pl.BlockSpec((_BLK, 1), lambda i: (i, 0))]
    args = [p, degp, dcorr]
    if bias is not None:
        in_specs.append(pl.BlockSpec((1, f), lambda i: (0, 0)))
        args.append(bias)
    return pl.pallas_call(
        body,
        grid=(n // _BLK,),
        in_specs=in_specs,
        out_specs=pl.BlockSpec((_BLK, f), lambda i: (i, 0)),
        out_shape=jax.ShapeDtypeStruct((n, f), jnp.float32),
    )(*args)


def _tc_combine_elu_mm(q, degp, dcorr, b, w):
    f = q.shape[2]
    n = degp.shape[1]
    f2 = w.shape[1]

    def body(q_r, d_r, c_r, b_r, w_r, o_r):
        ssum = q_r[0] + q_r[1]
        deg = d_r[0] + d_r[1] - c_r[...]
        h = ssum / jnp.maximum(deg, 1.0) + b_r[...]
        h = jnp.where(h > 0, h, jnp.exp(jnp.minimum(h, 0.0)) - 1.0)
        o_r[...] = jnp.dot(h, w_r[...], preferred_element_type=jnp.float32)

    return pl.pallas_call(
        body,
        grid=(n // _BLK,),
        in_specs=[pl.BlockSpec((NC, _BLK, f), lambda i: (0, i, 0)),
                  pl.BlockSpec((NC, _BLK, 1), lambda i: (0, i, 0)),
                  pl.BlockSpec((_BLK, 1), lambda i: (i, 0)),
                  pl.BlockSpec((1, f), lambda i: (0, 0)),
                  pl.BlockSpec((f, f2), lambda i: (0, 0))],
        out_specs=pl.BlockSpec((_BLK, f2), lambda i: (i, 0)),
        out_shape=jax.ShapeDtypeStruct((n, f2), jnp.float32),
    )(q, degp, dcorr, b, w)


def kernel(x, edge_index, W1, b1, W2, b2):
    n, _ = x.shape
    m = NUM_HYPEREDGES
    row = edge_index[0]
    col = edge_index[1]
    e = row.shape[0]

    C1, C2 = 88, 128

    n_pad_rows = NS * (-(-n // NS // 8) * 8) - n

    def padded(chunk):
        epad = NW * chunk * (-(-e // (NW * chunk)))
        pad = epad - e
        assert pad <= n
        ar = jnp.arange(pad, dtype=jnp.int32)
        gather_pad = ar
        scatter_pad = (n + ar % n_pad_rows).astype(jnp.int32)
        cat = lambda a, v: jnp.concatenate([a, v])
        return (cat(row, gather_pad), cat(row, scatter_pad),
                cat(col, gather_pad), cat(col, scatter_pad), pad)

    row_g1, row_s1, col_g1, col_s1, pad1 = padded(C1)
    row_g2, row_s2, col_g2, col_s2, _ = padded(C2)
    dcorr = (jnp.arange(n)[:, None] < pad1).astype(jnp.float32)

    xw = _tc_mm(x, W1)
    p1, dg, bg = _sc_propagate(xw, row_g1, col_s1, m, True, C1, nb=4, ahead=2)
    dg3 = dg.reshape(NC, -1)[:, :n].reshape(NC, n, 1)
    bg3 = bg.reshape(NC, -1)[:, :m].reshape(NC, m, 1)
    bcorr = jnp.zeros((m, 1), jnp.float32)
    out_e = _tc_combine_scale(p1, bg3, bcorr)
    (q1,) = _sc_propagate(out_e, col_g1, row_s1, n, False, C1, nb=4, ahead=2)
    h2 = _tc_combine_elu_mm(q1, dg3, dcorr, b1.reshape(1, -1), W2)

    (p2,) = _sc_propagate(h2, row_g2, col_s2, m, False, C2, nb=4, ahead=2)
    out_e2 = _tc_combine_scale(p2, bg3, bcorr)
    (q2,) = _sc_propagate(out_e2, col_g2, row_s2, n, False, C2, nb=4, ahead=2)
    out = _tc_combine_scale(q2, dg3, dcorr, bias=b2.reshape(1, -1))
    return out


# speedup vs baseline: 1.3079x; 1.3079x over previous
import jax
import jax.numpy as jnp
from jax import lax
from jax.experimental import pallas as pl
from jax.experimental.pallas import tpu as pltpu
from jax.experimental.pallas import tpu_sc as plsc

NC = 2
NS = 16
NW = NC * NS
NUM_HYPEREDGES = 10000


def _sc_propagate(table, gidx, sidx, n_out, with_deg, chunk, nb=3, ahead=1):
    E = gidx.shape[0]
    T, F = table.shape
    assert E % (NW * chunk) == 0
    ept = E // NW
    full = ept // chunk
    rpt = -(-n_out // NS // 8) * 8
    n_pad = NS * rpt
    assert n_pad > n_out
    dpt = -(-T // NS // 16) * 16 if with_deg else 0
    bpt = -(-(n_out + 1) // NS // 16) * 16 if with_deg else 0
    t_pad, b_pad = NS * dpt, NS * bpt
    zlen = max(dpt, bpt)

    out_type = [jax.ShapeDtypeStruct((NC, n_pad, F), jnp.float32)]
    if with_deg:
        out_type += [jax.ShapeDtypeStruct((NC * t_pad,), jnp.float32),
                     jax.ShapeDtypeStruct((NC * b_pad,), jnp.float32)]

    assert not with_deg or chunk % 16 == 0
    NB = nb
    NI = NB + 2
    scratch = []
    for _ in range(NB):
        scratch += [pltpu.VMEM((chunk, F), jnp.float32),
                    pltpu.SemaphoreType.DMA,
                    pltpu.SemaphoreType.DMA]
    for _ in range(NI):
        scratch += [pltpu.VMEM((chunk,), jnp.int32),
                    pltpu.VMEM((chunk,), jnp.int32),
                    pltpu.SemaphoreType.DMA]
    scratch.append(pltpu.VMEM_SHARED((n_pad, F), jnp.float32))
    if with_deg:
        scratch += [pltpu.VMEM((chunk,), jnp.float32),
                    pltpu.VMEM((zlen,), jnp.float32),
                    pltpu.VMEM_SHARED((t_pad,), jnp.float32),
                    pltpu.VMEM_SHARED((b_pad,), jnp.float32)]

    mesh = plsc.VectorSubcoreMesh(core_axis_name="c", subcore_axis_name="s",
                                  num_cores=NC, num_subcores=NS)

    def body(*refs):
        it = iter(refs)
        table_r, gidx_r, sidx_r, z2_r = (next(it) for _ in range(4))
        p_r = next(it)
        if with_deg:
            dg_r, bg_r = next(it), next(it)
        bufs = tuple(tuple(next(it) for _ in range(3)) for _ in range(NB))
        ibufs = tuple(tuple(next(it) for _ in range(3)) for _ in range(NI))
        acc = next(it)
        if with_deg:
            ones, zbuf, acc_d, acc_b = next(it), next(it), next(it), next(it)

        c = lax.axis_index("c")
        s = lax.axis_index("s")
        wid = c * NS + s
        base = wid * ept

        pltpu.sync_copy(z2_r, acc.at[pl.ds(s * rpt, rpt)])
        if with_deg:
            for i in range(zlen // 16):
                zbuf[pl.ds(i * 16, 16)] = jnp.zeros((16,), jnp.float32)
            pltpu.sync_copy(zbuf.at[pl.ds(0, dpt)], acc_d.at[pl.ds(s * dpt, dpt)])
            pltpu.sync_copy(zbuf.at[pl.ds(0, bpt)], acc_b.at[pl.ds(s * bpt, bpt)])
            for i in range(chunk // 16):
                ones[pl.ds(i * 16, 16)] = jnp.ones((16,), jnp.float32)
        plsc.subcore_barrier()

        def idx_slices(j):
            off = pl.multiple_of(base + j * chunk, 8)
            return gidx_r.at[pl.ds(off, chunk)], sidx_r.at[pl.ds(off, chunk)]

        def load_idx_sync(j, ib):
            gsl, ssl = idx_slices(j)
            pltpu.sync_copy(gsl, ib[0])
            pltpu.sync_copy(ssl, ib[1])

        def prefetch_idx(j, ib):
            gsl, ssl = idx_slices(j)
            pltpu.async_copy(gsl, ib[0], ib[2])
            pltpu.async_copy(ssl, ib[1], ib[2])

        def wait_idx(j, ib):
            gsl, ssl = idx_slices(j)
            pltpu.make_async_copy(gsl, ib[0], ib[2]).wait()
            pltpu.make_async_copy(ssl, ib[1], ib[2]).wait()

        A = ahead

        def step(j, bx, bz, ix, iz, ip):
            rowsx, gsemx, ssemx = bx
            rowsz, gsemz, ssemz = bz

            @pl.when(j >= NB - A)
            def _():
                pltpu.make_async_copy(rowsz, acc.at[iz[1]], ssemz).wait()

            @pl.when(j + A + 1 < full)
            def _():
                prefetch_idx(j + A + 1, ip)

            @pl.when(j + A < full)
            def _():
                wait_idx(j + A, iz)
                pltpu.async_copy(table_r.at[iz[0]], rowsz, gsemz)

            pltpu.make_async_copy(table_r.at[ix[0]], rowsx, gsemx).wait()

            pltpu.async_copy(rowsx, acc.at[ix[1]], ssemx, add=True)
            if with_deg:
                pltpu.sync_copy(ones, acc_d.at[ix[0]], add=True)
                pltpu.sync_copy(ones, acc_b.at[ix[1]], add=True)

        for i in range(min(A, full)):
            load_idx_sync(i, ibufs[i])
            pltpu.async_copy(table_r.at[ibufs[i][0]], bufs[i][0], bufs[i][1])
        if A < full:
            prefetch_idx(A, ibufs[A % NI])

        def loop_body(k, carry):
            j = NB * NI * k
            for t in range(NB * NI):
                jt = j + t
                step(jt, bufs[t % NB], bufs[(t + A) % NB],
                     ibufs[t % NI], ibufs[(t + A) % NI],
                     ibufs[(t + A + 1) % NI])
            return carry

        G = NB * NI
        lax.fori_loop(0, full // G, loop_body, 0)
        for t in range(full - G * (full // G)):
            j = G * (full // G) + t
            step(jnp.int32(j), bufs[j % NB], bufs[(j + A) % NB],
                 ibufs[j % NI], ibufs[(j + A) % NI],
                 ibufs[(j + A + 1) % NI])

        for t in range(max(0, full - (NB - A)), full):
            rowsL, _gs, ssemL = bufs[t % NB]
            pltpu.make_async_copy(rowsL, acc.at[ibufs[t % NI][1]], ssemL).wait()

        plsc.subcore_barrier()
        pltpu.sync_copy(acc.at[pl.ds(s * rpt, rpt)], p_r.at[c, pl.ds(s * rpt, rpt)])
        if with_deg:
            pltpu.sync_copy(acc_d.at[pl.ds(s * dpt, dpt)], zbuf.at[pl.ds(0, dpt)])
            pltpu.sync_copy(zbuf.at[pl.ds(0, dpt)],
                            dg_r.at[pl.ds(pl.multiple_of(c * t_pad + s * dpt, 8), dpt)])
            pltpu.sync_copy(acc_b.at[pl.ds(s * bpt, bpt)], zbuf.at[pl.ds(0, bpt)])
            pltpu.sync_copy(zbuf.at[pl.ds(0, bpt)],
                            bg_r.at[pl.ds(pl.multiple_of(c * b_pad + s * bpt, 8), bpt)])

    args = [table, gidx, sidx, jnp.zeros((rpt, F), jnp.float32)]

    f = pl.kernel(body, out_type=tuple(out_type), mesh=mesh,
                  scratch_types=scratch,
                  compiler_params=pltpu.CompilerParams(use_tc_tiling_on_sc=False))
    out = f(*args)
    return list(out) if isinstance(out, (tuple, list)) else [out]


_BLK = 1000


def _tc_mm(x, w):
    n, f = x.shape
    f2 = w.shape[1]

    def body(x_r, w_r, o_r):
        o_r[...] = jnp.dot(x_r[...], w_r[...], preferred_element_type=jnp.float32)

    return pl.pallas_call(
        body,
        grid=(n // _BLK,),
        in_specs=[pl.BlockSpec((_BLK, f), lambda i: (i, 0)),
                  pl.BlockSpec((f, f2), lambda i: (0, 0))],
        out_specs=pl.BlockSpec((_BLK, f2), lambda i: (i, 0)),
        out_shape=jax.ShapeDtypeStruct((n, f2), jnp.float32),
    )(x, w)


def _tc_combine_scale(p, degp, dcorr, bias=None):
    f = p.shape[2]
    n = degp.shape[1]

    def body(p_r, d_r, c_r, *rest):
        o_r = rest[-1]
        ssum = p_r[0] + p_r[1]
        deg = d_r[0] + d_r[1] - c_r[...]
        o = ssum / jnp.maximum(deg, 1.0)
        if bias is not None:
            o = o + rest[0][...]
        o_r[...] = o

    in_specs = [pl.BlockSpec((NC, _BLK, f), lambda i: (0, i, 0)),
                pl.BlockSpec((NC, _BLK, 1), lambda i: (0, i, 0)),
                pl.BlockSpec((_BLK, 1), lambda i: (i, 0))]
    args = [p, degp, dcorr]
    if bias is not None:
        in_specs.append(pl.BlockSpec((1, f), lambda i: (0, 0)))
        args.append(bias)
    return pl.pallas_call(
        body,
        grid=(n // _BLK,),
        in_specs=in_specs,
        out_specs=pl.BlockSpec((_BLK, f), lambda i: (i, 0)),
        out_shape=jax.ShapeDtypeStruct((n, f), jnp.float32),
    )(*args)


def _tc_combine_elu_mm(q, degp, dcorr, b, w):
    f = q.shape[2]
    n = degp.shape[1]
    f2 = w.shape[1]

    def body(q_r, d_r, c_r, b_r, w_r, o_r):
        ssum = q_r[0] + q_r[1]
        deg = d_r[0] + d_r[1] - c_r[...]
        h = ssum / jnp.maximum(deg, 1.0) + b_r[...]
        h = jnp.where(h > 0, h, jnp.exp(jnp.minimum(h, 0.0)) - 1.0)
        o_r[...] = jnp.dot(h, w_r[...], preferred_element_type=jnp.float32)

    return pl.pallas_call(
        body,
        grid=(n // _BLK,),
        in_specs=[pl.BlockSpec((NC, _BLK, f), lambda i: (0, i, 0)),
                  pl.BlockSpec((NC, _BLK, 1), lambda i: (0, i, 0)),
                  pl.BlockSpec((_BLK, 1), lambda i: (i, 0)),
                  pl.BlockSpec((1, f), lambda i: (0, 0)),
                  pl.BlockSpec((f, f2), lambda i: (0, 0))],
        out_specs=pl.BlockSpec((_BLK, f2), lambda i: (i, 0)),
        out_shape=jax.ShapeDtypeStruct((n, f2), jnp.float32),
    )(q, degp, dcorr, b, w)


def kernel(x, edge_index, W1, b1, W2, b2):
    n, _ = x.shape
    m = NUM_HYPEREDGES
    row = edge_index[0]
    col = edge_index[1]
    e = row.shape[0]

    C1, C2 = 112, 128

    n_pad_rows = NS * (-(-n // NS // 8) * 8) - n

    def padded(chunk):
        epad = NW * chunk * (-(-e // (NW * chunk)))
        pad = epad - e
        assert pad <= n
        ar = jnp.arange(pad, dtype=jnp.int32)
        gather_pad = ar
        scatter_pad = (n + ar % n_pad_rows).astype(jnp.int32)
        cat = lambda a, v: jnp.concatenate([a, v])
        return (cat(row, gather_pad), cat(row, scatter_pad),
                cat(col, gather_pad), cat(col, scatter_pad), pad)

    row_g1, row_s1, col_g1, col_s1, pad1 = padded(C1)
    row_g2, row_s2, col_g2, col_s2, _ = padded(C2)
    dcorr = (jnp.arange(n)[:, None] < pad1).astype(jnp.float32)

    xw = _tc_mm(x, W1)
    p1, dg, bg = _sc_propagate(xw, row_g1, col_s1, m, True, C1)
    dg3 = dg.reshape(NC, -1)[:, :n].reshape(NC, n, 1)
    bg3 = bg.reshape(NC, -1)[:, :m].reshape(NC, m, 1)
    bcorr = jnp.zeros((m, 1), jnp.float32)
    out_e = _tc_combine_scale(p1, bg3, bcorr)
    (q1,) = _sc_propagate(out_e, col_g1, row_s1, n, False, C1)
    h2 = _tc_combine_elu_mm(q1, dg3, dcorr, b1.reshape(1, -1), W2)

    (p2,) = _sc_propagate(h2, row_g2, col_s2, m, False, C2)
    out_e2 = _tc_combine_scale(p2, bg3, bcorr)
    (q2,) = _sc_propagate(out_e2, col_g2, row_s2, n, False, C2)
    out = _tc_combine_scale(q2, dg3, dcorr, bias=b2.reshape(1, -1))
    return out


# speedup vs baseline: 1.3431x; 1.0269x over previous
import jax
import jax.numpy as jnp
from jax import lax
from jax.experimental import pallas as pl
from jax.experimental.pallas import tpu as pltpu
from jax.experimental.pallas import tpu_sc as plsc

NC = 2
NS = 16
NW = NC * NS
NUM_HYPEREDGES = 10000


def _sc_propagate(table, gidx, sidx, n_out, with_deg, chunk, nb=3, ahead=1):
    E = gidx.shape[0]
    T, F = table.shape
    assert E % (NW * chunk) == 0
    ept = E // NW
    full = ept // chunk
    rpt = -(-n_out // NS // 8) * 8
    n_pad = NS * rpt
    assert n_pad > n_out
    dpt = -(-T // NS // 16) * 16 if with_deg else 0
    bpt = -(-(n_out + 1) // NS // 16) * 16 if with_deg else 0
    t_pad, b_pad = NS * dpt, NS * bpt
    zlen = max(dpt, bpt)

    out_type = [jax.ShapeDtypeStruct((NC, n_pad, F), jnp.float32)]
    if with_deg:
        out_type += [jax.ShapeDtypeStruct((NC * t_pad,), jnp.float32),
                     jax.ShapeDtypeStruct((NC * b_pad,), jnp.float32)]

    assert not with_deg or chunk % 16 == 0
    NB = nb
    NI = NB + 2
    scratch = []
    for _ in range(NB):
        scratch += [pltpu.VMEM((chunk, F), jnp.float32),
                    pltpu.SemaphoreType.DMA,
                    pltpu.SemaphoreType.DMA]
    for _ in range(NI):
        scratch += [pltpu.VMEM((chunk,), jnp.int32),
                    pltpu.VMEM((chunk,), jnp.int32),
                    pltpu.SemaphoreType.DMA]
    scratch.append(pltpu.VMEM_SHARED((n_pad, F), jnp.float32))
    if with_deg:
        scratch += [pltpu.VMEM((chunk,), jnp.float32),
                    pltpu.VMEM((zlen,), jnp.float32),
                    pltpu.VMEM_SHARED((t_pad,), jnp.float32),
                    pltpu.VMEM_SHARED((b_pad,), jnp.float32)]

    mesh = plsc.VectorSubcoreMesh(core_axis_name="c", subcore_axis_name="s",
                                  num_cores=NC, num_subcores=NS)

    def body(*refs):
        it = iter(refs)
        table_r, gidx_r, sidx_r, z2_r = (next(it) for _ in range(4))
        p_r = next(it)
        if with_deg:
            dg_r, bg_r = next(it), next(it)
        bufs = tuple(tuple(next(it) for _ in range(3)) for _ in range(NB))
        ibufs = tuple(tuple(next(it) for _ in range(3)) for _ in range(NI))
        acc = next(it)
        if with_deg:
            ones, zbuf, acc_d, acc_b = next(it), next(it), next(it), next(it)

        c = lax.axis_index("c")
        s = lax.axis_index("s")
        wid = c * NS + s
        base = wid * ept

        pltpu.sync_copy(z2_r, acc.at[pl.ds(s * rpt, rpt)])
        if with_deg:
            for i in range(zlen // 16):
                zbuf[pl.ds(i * 16, 16)] = jnp.zeros((16,), jnp.float32)
            pltpu.sync_copy(zbuf.at[pl.ds(0, dpt)], acc_d.at[pl.ds(s * dpt, dpt)])
            pltpu.sync_copy(zbuf.at[pl.ds(0, bpt)], acc_b.at[pl.ds(s * bpt, bpt)])
            for i in range(chunk // 16):
                ones[pl.ds(i * 16, 16)] = jnp.ones((16,), jnp.float32)
        plsc.subcore_barrier()

        def idx_slices(j):
            off = pl.multiple_of(base + j * chunk, 8)
            return gidx_r.at[pl.ds(off, chunk)], sidx_r.at[pl.ds(off, chunk)]

        def load_idx_sync(j, ib):
            gsl, ssl = idx_slices(j)
            pltpu.sync_copy(gsl, ib[0])
            pltpu.sync_copy(ssl, ib[1])

        def prefetch_idx(j, ib):
            gsl, ssl = idx_slices(j)
            pltpu.async_copy(gsl, ib[0], ib[2])
            pltpu.async_copy(ssl, ib[1], ib[2])

        def wait_idx(j, ib):
            gsl, ssl = idx_slices(j)
            pltpu.make_async_copy(gsl, ib[0], ib[2]).wait()
            pltpu.make_async_copy(ssl, ib[1], ib[2]).wait()

        A = ahead

        def step(j, bx, bz, ix, iz, ip):
            rowsx, gsemx, ssemx = bx
            rowsz, gsemz, ssemz = bz

            @pl.when(j >= NB - A)
            def _():
                pltpu.make_async_copy(rowsz, acc.at[iz[1]], ssemz).wait()

            @pl.when(j + A + 1 < full)
            def _():
                prefetch_idx(j + A + 1, ip)

            @pl.when(j + A < full)
            def _():
                wait_idx(j + A, iz)
                pltpu.async_copy(table_r.at[iz[0]], rowsz, gsemz)

            pltpu.make_async_copy(table_r.at[ix[0]], rowsx, gsemx).wait()

            pltpu.async_copy(rowsx, acc.at[ix[1]], ssemx, add=True)
            if with_deg:
                pltpu.sync_copy(ones, acc_d.at[ix[0]], add=True)
                pltpu.sync_copy(ones, acc_b.at[ix[1]], add=True)

        for i in range(min(A, full)):
            load_idx_sync(i, ibufs[i])
            pltpu.async_copy(table_r.at[ibufs[i][0]], bufs[i][0], bufs[i][1])
        if A < full:
            prefetch_idx(A, ibufs[A % NI])

        def loop_body(k, carry):
            j = NB * NI * k
            for t in range(NB * NI):
                jt = j + t
                step(jt, bufs[t % NB], bufs[(t + A) % NB],
                     ibufs[t % NI], ibufs[(t + A) % NI],
                     ibufs[(t + A + 1) % NI])
            return carry

        G = NB * NI
        lax.fori_loop(0, full // G, loop_body, 0)
        for t in range(full - G * (full // G)):
            j = G * (full // G) + t
            step(jnp.int32(j), bufs[j % NB], bufs[(j + A) % NB],
                 ibufs[j % NI], ibufs[(j + A) % NI],
                 ibufs[(j + A + 1) % NI])

        for t in range(max(0, full - (NB - A)), full):
            rowsL, _gs, ssemL = bufs[t % NB]
            pltpu.make_async_copy(rowsL, acc.at[ibufs[t % NI][1]], ssemL).wait()

        plsc.subcore_barrier()
        pltpu.sync_copy(acc.at[pl.ds(s * rpt, rpt)], p_r.at[c, pl.ds(s * rpt, rpt)])
        if with_deg:
            pltpu.sync_copy(acc_d.at[pl.ds(s * dpt, dpt)], zbuf.at[pl.ds(0, dpt)])
            pltpu.sync_copy(zbuf.at[pl.ds(0, dpt)],
                            dg_r.at[pl.ds(pl.multiple_of(c * t_pad + s * dpt, 8), dpt)])
            pltpu.sync_copy(acc_b.at[pl.ds(s * bpt, bpt)], zbuf.at[pl.ds(0, bpt)])
            pltpu.sync_copy(zbuf.at[pl.ds(0, bpt)],
                            bg_r.at[pl.ds(pl.multiple_of(c * b_pad + s * bpt, 8), bpt)])

    args = [table, gidx, sidx, jnp.zeros((rpt, F), jnp.float32)]

    f = pl.kernel(body, out_type=tuple(out_type), mesh=mesh,
                  scratch_types=scratch,
                  compiler_params=pltpu.CompilerParams(use_tc_tiling_on_sc=False))
    out = f(*args)
    return list(out) if isinstance(out, (tuple, list)) else [out]


_BLK = 1000


def _tc_mm(x, w):
    n, f = x.shape
    f2 = w.shape[1]

    def body(x_r, w_r, o_r):
        o_r[...] = jnp.dot(x_r[...], w_r[...], preferred_element_type=jnp.float32)

    return pl.pallas_call(
        body,
        grid=(n // _BLK,),
        in_specs=[pl.BlockSpec((_BLK, f), lambda i: (i, 0)),
                  pl.BlockSpec((f, f2), lambda i: (0, 0))],
        out_specs=pl.BlockSpec((_BLK, f2), lambda i: (i, 0)),
        out_shape=jax.ShapeDtypeStruct((n, f2), jnp.float32),
    )(x, w)


def _tc_combine_scale(p, degp, dcorr, bias=None):
    f = p.shape[2]
    n = degp.shape[1]

    def body(p_r, d_r, c_r, *rest):
        o_r = rest[-1]
        ssum = p_r[0] + p_r[1]
        deg = d_r[0] + d_r[1] - c_r[...]
        o = ssum / jnp.maximum(deg, 1.0)
        if bias is not None:
            o = o + rest[0][...]
        o_r[...] = o

    in_specs = [pl.BlockSpec((NC, _BLK, f), lambda i: (0, i, 0)),
                pl.BlockSpec((NC, _BLK, 1), lambda i: (0, i, 0)),
                pl.BlockSpec((_BLK, 1), lambda i: (i, 0))]
    args = [p, degp, dcorr]
    if bias is not None:
        in_specs.append(pl.BlockSpec((1, f), lambda i: (0, 0)))
        args.append(bias)
    return pl.pallas_call(
        body,
        grid=(n // _BLK,),
        in_specs=in_specs,
        out_specs=pl.BlockSpec((_BLK, f), lambda i: (i, 0)),
        out_shape=jax.ShapeDtypeStruct((n, f), jnp.float32),
    )(*args)


def _tc_combine_elu_mm(q, degp, dcorr, b, w):
    f = q.shape[2]
    n = degp.shape[1]
    f2 = w.shape[1]

    def body(q_r, d_r, c_r, b_r, w_r, o_r):
        ssum = q_r[0] + q_r[1]
        deg = d_r[0] + d_r[1] - c_r[...]
        h = ssum / jnp.maximum(deg, 1.0) + b_r[...]
        h = jnp.where(h > 0, h, jnp.exp(jnp.minimum(h, 0.0)) - 1.0)
        o_r[...] = jnp.dot(h, w_r[...], preferred_element_type=jnp.float32)

    return pl.pallas_call(
        body,
        grid=(n // _BLK,),
        in_specs=[pl.BlockSpec((NC, _BLK, f), lambda i: (0, i, 0)),
                  pl.BlockSpec((NC, _BLK, 1), lambda i: (0, i, 0)),
                  pl.BlockSpec((_BLK, 1), lambda i: (i, 0)),
                  pl.BlockSpec((1, f), lambda i: (0, 0)),
                  pl.BlockSpec((f, f2), lambda i: (0, 0))],
        out_specs=pl.BlockSpec((_BLK, f2), lambda i: (i, 0)),
        out_shape=jax.ShapeDtypeStruct((n, f2), jnp.float32),
    )(q, degp, dcorr, b, w)


def kernel(x, edge_index, W1, b1, W2, b2):
    n, _ = x.shape
    m = NUM_HYPEREDGES
    row = edge_index[0]
    col = edge_index[1]
    e = row.shape[0]

    C1, C2 = 112, 128

    n_pad_rows = NS * (-(-n // NS // 8) * 8) - n

    def padded(chunk):
        epad = NW * chunk * (-(-e // (NW * chunk)))
        pad = epad - e
        assert pad <= n
        ar = jnp.arange(pad, dtype=jnp.int32)
        gather_pad = ar
        scatter_pad = (n + ar % n_pad_rows).astype(jnp.int32)
        cat = lambda a, v: jnp.concatenate([a, v])
        return (cat(row, gather_pad), cat(row, scatter_pad),
                cat(col, gather_pad), cat(col, scatter_pad), pad)

    row_g1, row_s1, col_g1, col_s1, pad1 = padded(C1)
    row_g2, row_s2, col_g2, col_s2, _ = padded(C2)
    dcorr = (jnp.arange(n)[:, None] < pad1).astype(jnp.float32)

    xw = _tc_mm(x, W1)
    p1, dg, bg = _sc_propagate(xw, row_g1, col_s1, m, True, C1, ahead=2)
    dg3 = dg.reshape(NC, -1)[:, :n].reshape(NC, n, 1)
    bg3 = bg.reshape(NC, -1)[:, :m].reshape(NC, m, 1)
    bcorr = jnp.zeros((m, 1), jnp.float32)
    out_e = _tc_combine_scale(p1, bg3, bcorr)
    (q1,) = _sc_propagate(out_e, col_g1, row_s1, n, False, C1, ahead=2)
    h2 = _tc_combine_elu_mm(q1, dg3, dcorr, b1.reshape(1, -1), W2)

    (p2,) = _sc_propagate(h2, row_g2, col_s2, m, False, C2, ahead=2)
    out_e2 = _tc_combine_scale(p2, bg3, bcorr)
    (q2,) = _sc_propagate(out_e2, col_g2, row_s2, n, False, C2, ahead=2)
    out = _tc_combine_scale(q2, dg3, dcorr, bias=b2.reshape(1, -1))
    return out
